# Initial kernel scaffold; baseline (speedup 1.0000x reference)
import functools

import jax
import jax.numpy as jnp
from jax import lax
from jax.experimental import pallas as pl
from jax.experimental.pallas import tpu as pltpu
from jax.experimental.pallas import tpu_sc as plsc

B, N, C, K = 2, 4096, 256, 16
D_TAB = 272


T1 = 128


def _knn_body(p2_ref, p1_ref, idx_ref):
    b = pl.program_id(0)
    a = p2_ref[0]
    db = p1_ref[0]
    sa = jnp.sum(a * a, axis=0)[:, None]
    sb = jnp.sum(db * db, axis=0)[None, :]
    ab = lax.dot_general(a, db, (((0,), (0,)), ((), ())),
                         preferred_element_type=jnp.float32)
    d2 = sa + sb - 2.0 * ab
    iota = lax.broadcasted_iota(jnp.int32, (T1, N), 1)
    base = b * N
    vals = d2
    for j in range(K):
        m = jnp.min(vals, axis=1, keepdims=True)
        idxj = jnp.min(jnp.where(vals == m, iota, N), axis=1, keepdims=True)
        idx_ref[0, :, j:j + 1] = idxj + base
        vals = jnp.where(iota == idxj, jnp.inf, vals)


def _knn(points2, points1):
    return pl.pallas_call(
        _knn_body,
        grid=(B, N // T1),
        in_specs=[
            pl.BlockSpec((1, 3, T1), lambda b, i: (b, 0, i)),
            pl.BlockSpec((1, 3, N), lambda b, i: (b, 0, 0)),
        ],
        out_specs=pl.BlockSpec((1, T1, K), lambda b, i: (b, i, 0)),
        out_shape=jax.ShapeDtypeStruct((B, N, K), jnp.int32),
    )(points2, points1)


CHUNK = 128


def _make_sc_gather(btot, d):
    nc, ns = 2, 16
    nw = nc * ns
    bpw = btot // nw
    nch = bpw // CHUNK
    mesh = plsc.VectorSubcoreMesh(core_axis_name="c", subcore_axis_name="s",
                                  num_cores=nc, num_subcores=ns)

    @functools.partial(
        pl.kernel, mesh=mesh,
        out_type=jax.ShapeDtypeStruct((btot, d), jnp.float32),
        scratch_types=[
            pltpu.VMEM((CHUNK,), jnp.int32),
            pltpu.VMEM((CHUNK, d), jnp.float32),
            pltpu.SemaphoreType.DMA,
        ],
    )
    def gather(table_hbm, idx_hbm, out_hbm, idx_v, rows_v, sem):
        wid = lax.axis_index("s") * nc + lax.axis_index("c")
        base = wid * bpw

        def body(i, carry):
            off = base + i * CHUNK
            pltpu.sync_copy(idx_hbm.at[pl.ds(off, CHUNK)], idx_v)
            pltpu.async_copy(table_hbm.at[idx_v], rows_v, sem).wait()
            pltpu.sync_copy(rows_v, out_hbm.at[pl.ds(off, CHUNK)])
            return carry

        lax.fori_loop(0, nch, body, 0)

    return gather


T3 = 64


def _mlp_body(y_ref, a_ref, w1tg_ref, w1tm_ref, b1_ref, w2t_ref, b2_ref,
              out_ref):
    y = y_ref[0].reshape(T3 * K, D_TAB)
    mot = y[:, :C]
    p = y[:, C:C + 3]
    a = a_ref[0]
    a_rep = jnp.broadcast_to(a[:, None, :], (T3, K, 3)).reshape(T3 * K, 3)
    rela = p - a_rep
    d2e = jnp.sum(rela * rela, axis=1, keepdims=True)
    dist = jnp.sqrt(d2e)
    xg = jnp.concatenate([rela, dist], axis=1)
    h = (lax.dot_general(mot, w1tm_ref[...], (((1,), (0,)), ((), ())),
                         preferred_element_type=jnp.float32)
         + lax.dot_general(xg, w1tg_ref[...], (((1,), (0,)), ((), ())),
                           preferred_element_type=jnp.float32)
         + b1_ref[...])
    h2 = (lax.dot_general(h, w2t_ref[...], (((1,), (0,)), ((), ())),
                          preferred_element_type=jnp.float32)
          + b2_ref[...])
    wmax = jnp.max(h2, axis=1)
    wk = wmax.reshape(T3, K)
    wk = wk - jnp.max(wk, axis=1, keepdims=True)
    e = jnp.exp(wk)
    sm = e / jnp.sum(e, axis=1, keepdims=True)
    contrib = mot.reshape(T3, K, C) * sm[:, :, None]
    out_ref[0] = jnp.sum(contrib, axis=1)


def _mlp(y, anchors, w1tg, w1tm, b1r, w2t, b2r):
    return pl.pallas_call(
        _mlp_body,
        grid=(B, N // T3),
        in_specs=[
            pl.BlockSpec((1, T3, K, D_TAB), lambda b, i: (b, i, 0, 0)),
            pl.BlockSpec((1, T3, 3), lambda b, i: (b, i, 0)),
            pl.BlockSpec((4, C // 2), lambda b, i: (0, 0)),
            pl.BlockSpec((C, C // 2), lambda b, i: (0, 0)),
            pl.BlockSpec((1, C // 2), lambda b, i: (0, 0)),
            pl.BlockSpec((C // 2, C // 4), lambda b, i: (0, 0)),
            pl.BlockSpec((1, C // 4), lambda b, i: (0, 0)),
        ],
        out_specs=pl.BlockSpec((1, T3, C), lambda b, i: (b, i, 0)),
        out_shape=jax.ShapeDtypeStruct((B, N, C), jnp.float32),
    )(y, anchors, w1tg, w1tm, b1r, w2t, b2r)


def kernel(points1, points2, motion1, W1, b1, W2, b2):
    idx = _knn(points2, points1)

    p1t = jnp.transpose(points1, (0, 2, 1))
    p2t = jnp.transpose(points2, (0, 2, 1))
    mt = jnp.transpose(motion1, (0, 2, 1))
    pad = jnp.zeros((B, N, D_TAB - C - 3), jnp.float32)
    table = jnp.concatenate([mt, p1t, pad], axis=-1).reshape(B * N, D_TAB)

    y = _make_sc_gather(B * N * K, D_TAB)(table, idx.reshape(-1))
    y = y.reshape(B, N, K, D_TAB)

    w1tg = jnp.transpose(W1[:, :4])
    w1tm = jnp.transpose(W1[:, 4:])
    w2t = jnp.transpose(W2)
    out = _mlp(y, p2t, w1tg, w1tm, b1.reshape(1, -1), w2t, b2.reshape(1, -1))
    return jnp.transpose(out, (0, 2, 1))


# speedup vs baseline: 9.0043x; 9.0043x over previous
import functools

import jax
import jax.numpy as jnp
from jax import lax
from jax.experimental import pallas as pl
from jax.experimental.pallas import tpu as pltpu
from jax.experimental.pallas import tpu_sc as plsc

B, N, C, K = 2, 4096, 256, 16


T1 = 128


def _knn_body(p2_ref, p1_ref, idx_ref):
    b = pl.program_id(0)
    a = p2_ref[0]
    db = p1_ref[0]
    sa = jnp.sum(a * a, axis=0)[:, None]
    sb = jnp.sum(db * db, axis=0)[None, :]
    ab = lax.dot_general(a, db, (((0,), (0,)), ((), ())),
                         preferred_element_type=jnp.float32)
    d2 = sa + sb - 2.0 * ab
    iota = lax.broadcasted_iota(jnp.int32, (T1, N), 1)
    base = b * N
    vals = d2
    for j in range(K):
        m = jnp.min(vals, axis=1, keepdims=True)
        idxj = jnp.min(jnp.where(vals == m, iota, N), axis=1, keepdims=True)
        idx_ref[0, :, j:j + 1] = idxj + base
        vals = jnp.where(iota == idxj, jnp.inf, vals)


def _knn(points2, points1):
    return pl.pallas_call(
        _knn_body,
        grid=(B, N // T1),
        in_specs=[
            pl.BlockSpec((1, 3, T1), lambda b, i: (b, 0, i)),
            pl.BlockSpec((1, 3, N), lambda b, i: (b, 0, 0)),
        ],
        out_specs=pl.BlockSpec((1, T1, K), lambda b, i: (b, i, 0)),
        out_shape=jax.ShapeDtypeStruct((B, N, K), jnp.int32),
    )(points2, points1)


CHUNK = 128
NC, NS = 2, 16
NW = NC * NS
BTOT = B * N * K
BPW = BTOT // NW


def _sc_gather(table, px, py, pz, ax, ay, az, idx):
    nch = BPW // CHUNK
    mesh = plsc.VectorSubcoreMesh(core_axis_name="c", subcore_axis_name="s",
                                  num_cores=NC, num_subcores=NS)

    @functools.partial(
        pl.kernel, mesh=mesh,
        compiler_params=pltpu.CompilerParams(needs_layout_passes=False),
        out_type=(jax.ShapeDtypeStruct((BTOT, C), jnp.float32),
                  jax.ShapeDtypeStruct((BTOT,), jnp.float32),
                  jax.ShapeDtypeStruct((BTOT,), jnp.float32),
                  jax.ShapeDtypeStruct((BTOT,), jnp.float32),
                  jax.ShapeDtypeStruct((BTOT,), jnp.float32)),
        scratch_types=[
            pltpu.VMEM((CHUNK,), jnp.int32),
            pltpu.VMEM((CHUNK, C), jnp.float32),
            pltpu.VMEM((B * N,), jnp.float32),
            pltpu.VMEM((B * N,), jnp.float32),
            pltpu.VMEM((B * N,), jnp.float32),
            pltpu.VMEM((B * N,), jnp.float32),
            pltpu.VMEM((B * N,), jnp.float32),
            pltpu.VMEM((B * N,), jnp.float32),
            pltpu.VMEM((BPW,), jnp.float32),
            pltpu.VMEM((BPW,), jnp.float32),
            pltpu.VMEM((BPW,), jnp.float32),
            pltpu.VMEM((BPW,), jnp.float32),
            pltpu.SemaphoreType.DMA,
        ],
    )
    def gather(table_hbm, px_hbm, py_hbm, pz_hbm, ax_hbm, ay_hbm, az_hbm,
               idx_hbm, rows_out, xo_out, yo_out, zo_out, d2_out,
               idx_v, rows_v, px_v, py_v, pz_v, ax_v, ay_v, az_v,
               xv, yv, zv, d2v, sem):
        wid = lax.axis_index("s") * NC + lax.axis_index("c")
        base = wid * BPW
        pltpu.sync_copy(px_hbm, px_v)
        pltpu.sync_copy(py_hbm, py_v)
        pltpu.sync_copy(pz_hbm, pz_v)
        pltpu.sync_copy(ax_hbm, ax_v)
        pltpu.sync_copy(ay_hbm, ay_v)
        pltpu.sync_copy(az_hbm, az_v)

        def body(i, carry):
            off = base + i * CHUNK
            pltpu.sync_copy(idx_hbm.at[pl.ds(off, CHUNK)], idx_v)
            pltpu.async_copy(table_hbm.at[idx_v], rows_v, sem).wait()
            pltpu.sync_copy(rows_v, rows_out.at[pl.ds(off, CHUNK)])
            for j in range(CHUNK // 16):
                iv = idx_v[pl.ds(j * 16, 16)]
                dst = i * CHUNK + j * 16
                nvec = jnp.full((16,), off // K + j, jnp.int32)
                gx = plsc.load_gather(px_v, [iv]) - plsc.load_gather(ax_v, [nvec])
                gy = plsc.load_gather(py_v, [iv]) - plsc.load_gather(ay_v, [nvec])
                gz = plsc.load_gather(pz_v, [iv]) - plsc.load_gather(az_v, [nvec])
                xv[pl.ds(dst, 16)] = gx
                yv[pl.ds(dst, 16)] = gy
                zv[pl.ds(dst, 16)] = gz
                d2v[pl.ds(dst, 16)] = gx * gx + gy * gy + gz * gz
            return carry

        lax.fori_loop(0, nch, body, 0)
        pltpu.sync_copy(xv, xo_out.at[pl.ds(base, BPW)])
        pltpu.sync_copy(yv, yo_out.at[pl.ds(base, BPW)])
        pltpu.sync_copy(zv, zo_out.at[pl.ds(base, BPW)])
        pltpu.sync_copy(d2v, d2_out.at[pl.ds(base, BPW)])

    return gather(table, px, py, pz, ax, ay, az, idx)


T3 = 64
M3 = T3 * K


def _mlp_body(y_ref, x4_ref, w1tg_ref, w1tm_ref, b1_ref, w2t_ref,
              b2_ref, out_ref):
    mot = y_ref[0].reshape(M3, C)
    g = x4_ref[0]
    xg = jnp.concatenate([g[:, :3], jnp.sqrt(g[:, 3:4])], axis=1)
    h = (lax.dot_general(mot, w1tm_ref[...], (((1,), (0,)), ((), ())),
                         preferred_element_type=jnp.float32)
         + lax.dot_general(xg, w1tg_ref[...], (((1,), (0,)), ((), ())),
                           preferred_element_type=jnp.float32)
         + b1_ref[...])
    h2 = (lax.dot_general(h, w2t_ref[...], (((1,), (0,)), ((), ())),
                          preferred_element_type=jnp.float32)
          + b2_ref[...])
    wmax = jnp.max(h2, axis=1)
    wk = wmax.reshape(T3, K)
    wk = wk - jnp.max(wk, axis=1, keepdims=True)
    e = jnp.exp(wk)
    sm = e / jnp.sum(e, axis=1, keepdims=True)
    contrib = mot.reshape(T3, K, C) * sm[:, :, None]
    out_ref[0] = jnp.sum(contrib, axis=1)


def _mlp(y, x4, w1tg, w1tm, b1r, w2t, b2r):
    return pl.pallas_call(
        _mlp_body,
        grid=(B, N // T3),
        in_specs=[
            pl.BlockSpec((1, T3, K, C), lambda b, i: (b, i, 0, 0)),
            pl.BlockSpec((1, M3, 4), lambda b, i: (b, i, 0)),
            pl.BlockSpec((4, C // 2), lambda b, i: (0, 0)),
            pl.BlockSpec((C, C // 2), lambda b, i: (0, 0)),
            pl.BlockSpec((1, C // 2), lambda b, i: (0, 0)),
            pl.BlockSpec((C // 2, C // 4), lambda b, i: (0, 0)),
            pl.BlockSpec((1, C // 4), lambda b, i: (0, 0)),
        ],
        out_specs=pl.BlockSpec((1, T3, C), lambda b, i: (b, i, 0)),
        out_shape=jax.ShapeDtypeStruct((B, N, C), jnp.float32),
    )(y, x4, w1tg, w1tm, b1r, w2t, b2r)


def kernel(points1, points2, motion1, W1, b1, W2, b2):
    idx = _knn(points2, points1)

    table = jnp.transpose(motion1, (0, 2, 1)).reshape(B * N, C)
    px = points1[:, 0, :].reshape(-1)
    py = points1[:, 1, :].reshape(-1)
    pz = points1[:, 2, :].reshape(-1)
    ax = points2[:, 0, :].reshape(-1)
    ay = points2[:, 1, :].reshape(-1)
    az = points2[:, 2, :].reshape(-1)

    y, xo, yo, zo, d2o = _sc_gather(table, px, py, pz, ax, ay, az,
                                    idx.reshape(-1))
    y = y.reshape(B, N, K, C)
    x4 = jnp.stack([xo, yo, zo, d2o], axis=1).reshape(B, N * K, 4)

    w1tg = jnp.transpose(W1[:, :4])
    w1tm = jnp.transpose(W1[:, 4:])
    w2t = jnp.transpose(W2)
    out = _mlp(y, x4, w1tg, w1tm, b1.reshape(1, -1), w2t,
               b2.reshape(1, -1))
    return jnp.transpose(out, (0, 2, 1))


# speedup vs baseline: 11.4722x; 1.2741x over previous
import functools

import jax
import jax.numpy as jnp
from jax import lax
from jax.experimental import pallas as pl
from jax.experimental.pallas import tpu as pltpu
from jax.experimental.pallas import tpu_sc as plsc

B, N, C, K = 2, 4096, 256, 16


T1 = 512


NCH = 32
CW = N // NCH


def _knn_body(p2_ref, p1_ref, idx_ref):
    b = pl.program_id(0)
    a = p2_ref[0]
    db = p1_ref[0]
    sa = jnp.sum(a * a, axis=0)[:, None]
    sb = jnp.sum(db * db, axis=0)[None, :]
    ab = lax.dot_general(a, db, (((0,), (0,)), ((), ())),
                         preferred_element_type=jnp.float32)
    d2 = sa + sb - 2.0 * ab
    chunks = [d2[:, c * CW:(c + 1) * CW] for c in range(NCH)]
    cmin = jnp.concatenate(
        [jnp.min(ch, axis=1, keepdims=True) for ch in chunks], axis=1)
    iota_c = lax.broadcasted_iota(jnp.int32, (T1, NCH), 1)
    iota_w = lax.broadcasted_iota(jnp.int32, (T1, CW), 1)
    inf = jnp.float32(jnp.inf)
    base = b * N
    sel = []
    for j in range(K):
        m = jnp.min(cmin, axis=1, keepdims=True)
        cidx = jnp.min(jnp.where(cmin == m, iota_c, NCH), axis=1,
                       keepdims=True)
        acc = chunks[NCH - 1]
        for c in range(NCH - 2, -1, -1):
            acc = jnp.where(cidx == c, chunks[c], acc)
        gcur = cidx * CW + iota_w
        for p in sel:
            acc = jnp.where(gcur == p, inf, acc)
        lidx = jnp.min(jnp.where(acc == m, iota_w, CW), axis=1,
                       keepdims=True)
        gidx = cidx * CW + lidx
        sel.append(gidx)
        idx_ref[0, :, j:j + 1] = gidx + base
        if j < K - 1:
            nmin = jnp.min(jnp.where(iota_w == lidx, inf, acc), axis=1,
                           keepdims=True)
            cmin = jnp.where(iota_c == cidx, nmin, cmin)


def _knn(points2, points1):
    return pl.pallas_call(
        _knn_body,
        grid=(B, N // T1),
        in_specs=[
            pl.BlockSpec((1, 3, T1), lambda b, i: (b, 0, i)),
            pl.BlockSpec((1, 3, N), lambda b, i: (b, 0, 0)),
        ],
        out_specs=pl.BlockSpec((1, T1, K), lambda b, i: (b, i, 0)),
        out_shape=jax.ShapeDtypeStruct((B, N, K), jnp.int32),
    )(points2, points1)


CHUNK = 128
NC, NS = 2, 16
NW = NC * NS
BTOT = B * N * K
BPW = BTOT // NW


def _sc_gather(table, px, py, pz, ax, ay, az, idx):
    nch = BPW // CHUNK
    mesh = plsc.VectorSubcoreMesh(core_axis_name="c", subcore_axis_name="s",
                                  num_cores=NC, num_subcores=NS)

    @functools.partial(
        pl.kernel, mesh=mesh,
        compiler_params=pltpu.CompilerParams(needs_layout_passes=False),
        out_type=(jax.ShapeDtypeStruct((BTOT, C), jnp.float32),
                  jax.ShapeDtypeStruct((BTOT,), jnp.float32),
                  jax.ShapeDtypeStruct((BTOT,), jnp.float32),
                  jax.ShapeDtypeStruct((BTOT,), jnp.float32),
                  jax.ShapeDtypeStruct((BTOT,), jnp.float32)),
        scratch_types=[
            pltpu.VMEM((CHUNK,), jnp.int32),
            pltpu.VMEM((CHUNK, C), jnp.float32),
            pltpu.VMEM((B * N,), jnp.float32),
            pltpu.VMEM((B * N,), jnp.float32),
            pltpu.VMEM((B * N,), jnp.float32),
            pltpu.VMEM((B * N,), jnp.float32),
            pltpu.VMEM((B * N,), jnp.float32),
            pltpu.VMEM((B * N,), jnp.float32),
            pltpu.VMEM((BPW,), jnp.float32),
            pltpu.VMEM((BPW,), jnp.float32),
            pltpu.VMEM((BPW,), jnp.float32),
            pltpu.VMEM((BPW,), jnp.float32),
            pltpu.SemaphoreType.DMA,
        ],
    )
    def gather(table_hbm, px_hbm, py_hbm, pz_hbm, ax_hbm, ay_hbm, az_hbm,
               idx_hbm, rows_out, xo_out, yo_out, zo_out, d2_out,
               idx_v, rows_v, px_v, py_v, pz_v, ax_v, ay_v, az_v,
               xv, yv, zv, d2v, sem):
        wid = lax.axis_index("s") * NC + lax.axis_index("c")
        base = wid * BPW
        pltpu.sync_copy(px_hbm, px_v)
        pltpu.sync_copy(py_hbm, py_v)
        pltpu.sync_copy(pz_hbm, pz_v)
        pltpu.sync_copy(ax_hbm, ax_v)
        pltpu.sync_copy(ay_hbm, ay_v)
        pltpu.sync_copy(az_hbm, az_v)

        def body(i, carry):
            off = base + i * CHUNK
            pltpu.sync_copy(idx_hbm.at[pl.ds(off, CHUNK)], idx_v)
            pltpu.async_copy(table_hbm.at[idx_v], rows_v, sem).wait()
            pltpu.sync_copy(rows_v, rows_out.at[pl.ds(off, CHUNK)])
            for j in range(CHUNK // 16):
                iv = idx_v[pl.ds(j * 16, 16)]
                dst = i * CHUNK + j * 16
                nvec = jnp.full((16,), off // K + j, jnp.int32)
                gx = plsc.load_gather(px_v, [iv]) - plsc.load_gather(ax_v, [nvec])
                gy = plsc.load_gather(py_v, [iv]) - plsc.load_gather(ay_v, [nvec])
                gz = plsc.load_gather(pz_v, [iv]) - plsc.load_gather(az_v, [nvec])
                xv[pl.ds(dst, 16)] = gx
                yv[pl.ds(dst, 16)] = gy
                zv[pl.ds(dst, 16)] = gz
                d2v[pl.ds(dst, 16)] = gx * gx + gy * gy + gz * gz
            return carry

        lax.fori_loop(0, nch, body, 0)
        pltpu.sync_copy(xv, xo_out.at[pl.ds(base, BPW)])
        pltpu.sync_copy(yv, yo_out.at[pl.ds(base, BPW)])
        pltpu.sync_copy(zv, zo_out.at[pl.ds(base, BPW)])
        pltpu.sync_copy(d2v, d2_out.at[pl.ds(base, BPW)])

    return gather(table, px, py, pz, ax, ay, az, idx)


T3 = 64
M3 = T3 * K


def _mlp_body(y_ref, x4_ref, w1tg_ref, w1tm_ref, b1_ref, w2t_ref,
              b2_ref, out_ref):
    mot = y_ref[0].reshape(M3, C)
    g = x4_ref[0]
    xg = jnp.concatenate([g[:, :3], jnp.sqrt(g[:, 3:4])], axis=1)
    h = (lax.dot_general(mot, w1tm_ref[...], (((1,), (0,)), ((), ())),
                         preferred_element_type=jnp.float32)
         + lax.dot_general(xg, w1tg_ref[...], (((1,), (0,)), ((), ())),
                           preferred_element_type=jnp.float32)
         + b1_ref[...])
    h2 = (lax.dot_general(h, w2t_ref[...], (((1,), (0,)), ((), ())),
                          preferred_element_type=jnp.float32)
          + b2_ref[...])
    wmax = jnp.max(h2, axis=1)
    wk = wmax.reshape(T3, K)
    wk = wk - jnp.max(wk, axis=1, keepdims=True)
    e = jnp.exp(wk)
    sm = e / jnp.sum(e, axis=1, keepdims=True)
    contrib = mot.reshape(T3, K, C) * sm[:, :, None]
    out_ref[0] = jnp.sum(contrib, axis=1)


def _mlp(y, x4, w1tg, w1tm, b1r, w2t, b2r):
    return pl.pallas_call(
        _mlp_body,
        grid=(B, N // T3),
        in_specs=[
            pl.BlockSpec((1, T3, K, C), lambda b, i: (b, i, 0, 0)),
            pl.BlockSpec((1, M3, 4), lambda b, i: (b, i, 0)),
            pl.BlockSpec((4, C // 2), lambda b, i: (0, 0)),
            pl.BlockSpec((C, C // 2), lambda b, i: (0, 0)),
            pl.BlockSpec((1, C // 2), lambda b, i: (0, 0)),
            pl.BlockSpec((C // 2, C // 4), lambda b, i: (0, 0)),
            pl.BlockSpec((1, C // 4), lambda b, i: (0, 0)),
        ],
        out_specs=pl.BlockSpec((1, T3, C), lambda b, i: (b, i, 0)),
        out_shape=jax.ShapeDtypeStruct((B, N, C), jnp.float32),
    )(y, x4, w1tg, w1tm, b1r, w2t, b2r)


def kernel(points1, points2, motion1, W1, b1, W2, b2):
    idx = _knn(points2, points1)

    table = jnp.transpose(motion1, (0, 2, 1)).reshape(B * N, C)
    px = points1[:, 0, :].reshape(-1)
    py = points1[:, 1, :].reshape(-1)
    pz = points1[:, 2, :].reshape(-1)
    ax = points2[:, 0, :].reshape(-1)
    ay = points2[:, 1, :].reshape(-1)
    az = points2[:, 2, :].reshape(-1)

    y, xo, yo, zo, d2o = _sc_gather(table, px, py, pz, ax, ay, az,
                                    idx.reshape(-1))
    y = y.reshape(B, N, K, C)
    x4 = jnp.stack([xo, yo, zo, d2o], axis=1).reshape(B, N * K, 4)

    w1tg = jnp.transpose(W1[:, :4])
    w1tm = jnp.transpose(W1[:, 4:])
    w2t = jnp.transpose(W2)
    out = _mlp(y, x4, w1tg, w1tm, b1.reshape(1, -1), w2t,
               b2.reshape(1, -1))
    return jnp.transpose(out, (0, 2, 1))
